# Initial kernel scaffold; baseline (speedup 1.0000x reference)
import jax
import jax.numpy as jnp
from jax.experimental import pallas as pl


def kernel(x, bessel_weights, r_values, bessel_values):
    raise NotImplementedError("write your pallas kernel here")


# speedup vs baseline: 209.5594x; 209.5594x over previous
import functools

import jax
import jax.numpy as jnp
from jax import lax
from jax.experimental import pallas as pl
from jax.experimental.pallas import tpu as pltpu
from jax.experimental.pallas import tpu_sc as plsc

_NUM_POINTS = 5000
_NUM_BASIS = 8
_R_MAX = 5.0
_B = 6400000

_NC = 2
_NS = 16
_NW = _NC * _NS
_PER_W = _B // _NW
_CHUNK = 2000
_N_CHUNKS = _PER_W // _CHUNK


def _tec_body(x_hbm, w2_hbm, r_hbm, tbl_hbm, out_hbm,
              tbl_v, r_v, w_v, x_v, out_v):
    wid = lax.axis_index("s") * _NC + lax.axis_index("c")
    base = wid * _PER_W

    pltpu.sync_copy(tbl_hbm, tbl_v)
    pltpu.sync_copy(r_hbm, r_v)
    pltpu.sync_copy(w2_hbm, w_v)

    wpat = w_v[...]
    lanes = lax.iota(jnp.int32, 16)
    pat8 = lanes * jnp.full((16,), _NUM_BASIS, jnp.int32)
    scat_idx = [pat8 + jnp.full((16,), col, jnp.int32)
                for col in range(_NUM_BASIS)]
    col_iv = [jnp.full((16,), col, jnp.int32) for col in range(_NUM_BASIS)]
    zero_i = jnp.zeros((16,), jnp.int32)
    one_i = jnp.ones((16,), jnp.int32)
    kmax_i = jnp.full((16,), _NUM_POINTS - 1, jnp.int32)
    nb_i = jnp.full((16,), _NUM_BASIS, jnp.int32)
    rmax_v = jnp.full((16,), _R_MAX, jnp.float32)
    inv_v = jnp.full((16,), (_NUM_POINTS - 1) / _R_MAX, jnp.float32)

    def _scale(i, c):
        sl = pl.ds(i * 16, 16)
        tbl_v[sl] = tbl_v[sl] * wpat
        return c
    lax.fori_loop(0, _NUM_POINTS * _NUM_BASIS // 16, _scale, 0)

    def _chunk(g, c):
        xoff = base + g * _CHUNK
        pltpu.sync_copy(x_hbm.at[pl.ds(xoff, _CHUNK)], x_v)

        def _group(j, cc):
            xv = x_v[pl.ds(j * 16, 16)]
            xc = jnp.minimum(xv, rmax_v)
            t = xc * inv_v
            kt = t.astype(jnp.int32)
            ktf = kt.astype(jnp.float32)
            k = kt + jnp.where(ktf < t, one_i, zero_i)
            k = jnp.minimum(jnp.maximum(k, zero_i), kmax_i)
            a = plsc.load_gather(r_v, [k])
            km = jnp.maximum(k - one_i, zero_i)
            b = plsc.load_gather(r_v, [km])
            kup = jnp.minimum(k + one_i, kmax_i)
            down = (b >= xc) & (k > zero_i)
            k = jnp.where(a < xc, kup, jnp.where(down, km, k))
            idx8 = k * nb_i
            oslice = out_v.at[pl.ds(j * (16 * _NUM_BASIS), 16 * _NUM_BASIS)]
            for col in range(_NUM_BASIS):
                v = plsc.load_gather(tbl_v, [idx8 + col_iv[col]])
                plsc.store_scatter(oslice, [scat_idx[col]], v)
            return cc
        lax.fori_loop(0, _CHUNK // 16, _group, 0)

        pltpu.sync_copy(out_v, out_hbm.at[pl.ds(xoff * _NUM_BASIS,
                                                _CHUNK * _NUM_BASIS)])
        return c
    lax.fori_loop(0, _N_CHUNKS, _chunk, 0)


_sc_call = pl.kernel(
    _tec_body,
    out_type=jax.ShapeDtypeStruct((_B * _NUM_BASIS,), jnp.float32),
    mesh=plsc.VectorSubcoreMesh(core_axis_name="c", subcore_axis_name="s"),
    compiler_params=pltpu.CompilerParams(needs_layout_passes=False),
    scratch_types=[
        pltpu.VMEM((_NUM_POINTS * _NUM_BASIS,), jnp.float32),
        pltpu.VMEM((_NUM_POINTS,), jnp.float32),
        pltpu.VMEM((16,), jnp.float32),
        pltpu.VMEM((_CHUNK,), jnp.float32),
        pltpu.VMEM((_CHUNK * _NUM_BASIS,), jnp.float32),
    ],
)


def kernel(x, bessel_weights, r_values, bessel_values):
    w2 = jnp.concatenate([bessel_weights, bessel_weights])
    tbl = bessel_values.reshape(-1)
    out = _sc_call(x, w2, r_values, tbl)
    return out.reshape(_B, _NUM_BASIS)


# speedup vs baseline: 210.3012x; 1.0035x over previous
import functools

import jax
import jax.numpy as jnp
from jax import lax
from jax.experimental import pallas as pl
from jax.experimental.pallas import tpu as pltpu
from jax.experimental.pallas import tpu_sc as plsc

_NUM_POINTS = 5000
_NUM_BASIS = 8
_R_MAX = 5.0
_B = 6400000

_NC = 2
_NS = 16
_NW = _NC * _NS
_PER_W = _B // _NW
_CHUNK = 2000
_N_CHUNKS = _PER_W // _CHUNK


def _tec_body(x_hbm, w2_hbm, r_hbm, tbl_hbm, out_hbm,
              tbl_v, r_v, w_v, x_v, out_v):
    wid = lax.axis_index("s") * _NC + lax.axis_index("c")
    base = wid * _PER_W

    pltpu.sync_copy(tbl_hbm, tbl_v)
    pltpu.sync_copy(r_hbm, r_v)
    pltpu.sync_copy(w2_hbm, w_v)

    wpat = w_v[...]
    lanes = lax.iota(jnp.int32, 16)
    pat8 = lanes * jnp.full((16,), _NUM_BASIS, jnp.int32)
    scat_idx = [pat8 + jnp.full((16,), col, jnp.int32)
                for col in range(_NUM_BASIS)]
    col_iv = [jnp.full((16,), col, jnp.int32) for col in range(_NUM_BASIS)]
    zero_i = jnp.zeros((16,), jnp.int32)
    one_i = jnp.ones((16,), jnp.int32)
    kmax_i = jnp.full((16,), _NUM_POINTS - 1, jnp.int32)
    nb_i = jnp.full((16,), _NUM_BASIS, jnp.int32)
    rmax_v = jnp.full((16,), _R_MAX, jnp.float32)
    inv_v = jnp.full((16,), (_NUM_POINTS - 1) / _R_MAX, jnp.float32)

    def _scale(i, c):
        sl = pl.ds(i * 16, 16)
        tbl_v[sl] = tbl_v[sl] * wpat
        return c
    lax.fori_loop(0, _NUM_POINTS * _NUM_BASIS // 16, _scale, 0, unroll=4)

    def _chunk(g, c):
        xoff = base + g * _CHUNK
        pltpu.sync_copy(x_hbm.at[pl.ds(xoff, _CHUNK)], x_v)

        def _group(j, cc):
            xv = x_v[pl.ds(j * 16, 16)]
            xc = jnp.minimum(xv, rmax_v)
            t = xc * inv_v
            kt = t.astype(jnp.int32)
            ktf = kt.astype(jnp.float32)
            k = kt + jnp.where(ktf < t, one_i, zero_i)
            k = jnp.minimum(jnp.maximum(k, zero_i), kmax_i)
            a = plsc.load_gather(r_v, [k])
            km = jnp.maximum(k - one_i, zero_i)
            b = plsc.load_gather(r_v, [km])
            kup = jnp.minimum(k + one_i, kmax_i)
            down = (b >= xc) & (k > zero_i)
            k = jnp.where(a < xc, kup, jnp.where(down, km, k))
            idx8 = k * nb_i
            oslice = out_v.at[pl.ds(j * (16 * _NUM_BASIS), 16 * _NUM_BASIS)]
            for col in range(_NUM_BASIS):
                v = plsc.load_gather(tbl_v, [idx8 + col_iv[col]])
                plsc.store_scatter(oslice, [scat_idx[col]], v)
            return cc
        lax.fori_loop(0, _CHUNK // 16, _group, 0, unroll=5)

        pltpu.sync_copy(out_v, out_hbm.at[pl.ds(xoff * _NUM_BASIS,
                                                _CHUNK * _NUM_BASIS)])
        return c
    lax.fori_loop(0, _N_CHUNKS, _chunk, 0)


_sc_call = pl.kernel(
    _tec_body,
    out_type=jax.ShapeDtypeStruct((_B * _NUM_BASIS,), jnp.float32),
    mesh=plsc.VectorSubcoreMesh(core_axis_name="c", subcore_axis_name="s"),
    compiler_params=pltpu.CompilerParams(needs_layout_passes=False),
    scratch_types=[
        pltpu.VMEM((_NUM_POINTS * _NUM_BASIS,), jnp.float32),
        pltpu.VMEM((_NUM_POINTS,), jnp.float32),
        pltpu.VMEM((16,), jnp.float32),
        pltpu.VMEM((_CHUNK,), jnp.float32),
        pltpu.VMEM((_CHUNK * _NUM_BASIS,), jnp.float32),
    ],
)


def kernel(x, bessel_weights, r_values, bessel_values):
    w2 = jnp.concatenate([bessel_weights, bessel_weights])
    tbl = bessel_values.reshape(-1)
    out = _sc_call(x, w2, r_values, tbl)
    return out.reshape(_B, _NUM_BASIS)


# speedup vs baseline: 763.8169x; 3.6320x over previous
import jax
import jax.numpy as jnp
from jax import lax
from jax.experimental import pallas as pl
from jax.experimental.pallas import tpu as pltpu
from jax.experimental.pallas import tpu_sc as plsc

_NUM_POINTS = 5000
_NUM_BASIS = 8
_R_MAX = 5.0
_B = 6400000

_NC = 2
_NS = 16
_NW = _NC * _NS
_CHUNK = 2048
_N_CHUNKS = _B // _CHUNK
_ROUNDS = -(-_N_CHUNKS // _NW)


def _tec_body(x_hbm, w2_hbm, r_hbm, tbl_hbm, out_hbm,
              tbl_v, r_v, w_v, x_v, out_v):
    wid = lax.axis_index("s") * _NC + lax.axis_index("c")

    pltpu.sync_copy(tbl_hbm, tbl_v)
    pltpu.sync_copy(r_hbm, r_v)
    pltpu.sync_copy(w2_hbm, w_v)

    wpat = w_v[...]
    zero_i = jnp.zeros((16,), jnp.int32)
    one_i = jnp.ones((16,), jnp.int32)
    kmax_i = jnp.full((16,), _NUM_POINTS - 1, jnp.int32)
    nb_i = jnp.full((16,), _NUM_BASIS, jnp.int32)
    rmax_v = jnp.full((16,), _R_MAX, jnp.float32)
    inv_v = jnp.full((16,), (_NUM_POINTS - 1) / _R_MAX, jnp.float32)
    col_iv = [jnp.full((16,), col, jnp.int32) for col in range(_NUM_BASIS)]

    def _scale(i, c):
        sl = pl.ds(i * 16, 16)
        tbl_v[sl] = tbl_v[sl] * wpat
        return c
    lax.fori_loop(0, _NUM_POINTS * _NUM_BASIS // 16, _scale, 0, unroll=4)

    def _round(i, c):
        ch = wid + i * _NW

        @pl.when(ch < _N_CHUNKS)
        def _do():
            pltpu.sync_copy(x_hbm.at[pl.ds(ch * _CHUNK, _CHUNK)], x_v)

            def _group(g, cc):
                xv = x_v[pl.ds(g * 16, 16)]
                xc = jnp.minimum(xv, rmax_v)
                t = xc * inv_v
                kt = t.astype(jnp.int32)
                ktf = kt.astype(jnp.float32)
                k = kt + jnp.where(ktf < t, one_i, zero_i)
                k = jnp.minimum(jnp.maximum(k, zero_i), kmax_i)
                a = plsc.load_gather(r_v, [k])
                km = jnp.maximum(k - one_i, zero_i)
                b = plsc.load_gather(r_v, [km])
                kup = jnp.minimum(k + one_i, kmax_i)
                down = (b >= xc) & (k > zero_i)
                k = jnp.where(a < xc, kup, jnp.where(down, km, k))
                idx8 = k * nb_i
                off = (g // 8) * (_NUM_BASIS * 128) + (g % 8) * 16
                for col in range(_NUM_BASIS):
                    v = plsc.load_gather(tbl_v, [idx8 + col_iv[col]])
                    out_v[pl.ds(off + col * 128, 16)] = v
                return cc
            lax.fori_loop(0, _CHUNK // 16, _group, 0, unroll=4)

            pltpu.sync_copy(
                out_v,
                out_hbm.at[pl.ds(ch * (_CHUNK * _NUM_BASIS),
                                 _CHUNK * _NUM_BASIS)])
        return c
    lax.fori_loop(0, _ROUNDS, _round, 0)


_sc_call = pl.kernel(
    _tec_body,
    out_type=jax.ShapeDtypeStruct((_B * _NUM_BASIS,), jnp.float32),
    mesh=plsc.VectorSubcoreMesh(core_axis_name="c", subcore_axis_name="s"),
    compiler_params=pltpu.CompilerParams(needs_layout_passes=False),
    scratch_types=[
        pltpu.VMEM((_NUM_POINTS * _NUM_BASIS,), jnp.float32),
        pltpu.VMEM((_NUM_POINTS,), jnp.float32),
        pltpu.VMEM((16,), jnp.float32),
        pltpu.VMEM((_CHUNK,), jnp.float32),
        pltpu.VMEM((_CHUNK * _NUM_BASIS,), jnp.float32),
    ],
)


def kernel(x, bessel_weights, r_values, bessel_values):
    w2 = jnp.concatenate([bessel_weights, bessel_weights])
    tbl = bessel_values.reshape(-1)
    out = _sc_call(x, w2, r_values, tbl)
    return out.reshape(_B // 128, _NUM_BASIS, 128).transpose(0, 2, 1).reshape(
        _B, _NUM_BASIS)


# speedup vs baseline: 878.1427x; 1.1497x over previous
import jax
import jax.numpy as jnp
from jax import lax
from jax.experimental import pallas as pl
from jax.experimental.pallas import tpu as pltpu
from jax.experimental.pallas import tpu_sc as plsc

_NUM_POINTS = 5000
_NUM_BASIS = 8
_R_MAX = 5.0
_B = 6400000

_NC = 2
_NS = 16
_NW = _NC * _NS
_CHUNK = 2048
_OUT_CHUNK = _CHUNK * _NUM_BASIS
_N_CHUNKS = _B // _CHUNK
_ROUNDS = -(-_N_CHUNKS // _NW)
assert _ROUNDS % 2 == 0


def _tec_body(x_hbm, w2_hbm, r_hbm, tbl_hbm, out_hbm,
              tbl_v, r_v, w_v, x_v0, x_v1, out_v0, out_v1,
              sx0, sx1, so0, so1):
    wid = lax.axis_index("s") * _NC + lax.axis_index("c")

    pltpu.sync_copy(tbl_hbm, tbl_v)
    pltpu.sync_copy(r_hbm, r_v)
    pltpu.sync_copy(w2_hbm, w_v)

    wpat = w_v[...]
    zero_i = jnp.zeros((16,), jnp.int32)
    one_i = jnp.ones((16,), jnp.int32)
    kmax_i = jnp.full((16,), _NUM_POINTS - 1, jnp.int32)
    nb_i = jnp.full((16,), _NUM_BASIS, jnp.int32)
    rmax_v = jnp.full((16,), _R_MAX, jnp.float32)
    inv_v = jnp.full((16,), (_NUM_POINTS - 1) / _R_MAX, jnp.float32)
    col_iv = [jnp.full((16,), col, jnp.int32) for col in range(_NUM_BASIS)]

    def _scale(i, c):
        sl = pl.ds(i * 16, 16)
        tbl_v[sl] = tbl_v[sl] * wpat
        return c
    lax.fori_loop(0, _NUM_POINTS * _NUM_BASIS // 16, _scale, 0, unroll=4)

    xbufs = (x_v0, x_v1)
    obufs = (out_v0, out_v1)
    xsems = (sx0, sx1)
    osems = (so0, so1)

    def _compute(xb, ob):
        def _group(g, cc):
            xv = xb[pl.ds(g * 16, 16)]
            xc = jnp.minimum(xv, rmax_v)
            t = xc * inv_v
            kt = t.astype(jnp.int32)
            ktf = kt.astype(jnp.float32)
            k = kt + jnp.where(ktf < t, one_i, zero_i)
            k = jnp.minimum(jnp.maximum(k, zero_i), kmax_i)
            a = plsc.load_gather(r_v, [k])
            km = jnp.maximum(k - one_i, zero_i)
            b = plsc.load_gather(r_v, [km])
            kup = jnp.minimum(k + one_i, kmax_i)
            down = (b >= xc) & (k > zero_i)
            k = jnp.where(a < xc, kup, jnp.where(down, km, k))
            idx8 = k * nb_i
            off = (g // 8) * (_NUM_BASIS * 128) + (g % 8) * 16
            for col in range(_NUM_BASIS):
                v = plsc.load_gather(tbl_v, [idx8 + col_iv[col]])
                ob[pl.ds(off + col * 128, 16)] = v
            return cc
        lax.fori_loop(0, _CHUNK // 16, _group, 0, unroll=4)

    pltpu.async_copy(x_hbm.at[pl.ds(wid * _CHUNK, _CHUNK)], x_v0, sx0)
    pltpu.async_copy(x_hbm.at[pl.ds((wid + _NW) * _CHUNK, _CHUNK)],
                     x_v1, sx1)

    def _round_pair(i, c):
        for p in range(2):
            r = i * 2 + p
            ch = wid + r * _NW
            xb, ob, sxb, sob = xbufs[p], obufs[p], xsems[p], osems[p]

            @pl.when(ch < _N_CHUNKS)
            def _do(ch=ch, xb=xb, ob=ob, sxb=sxb, sob=sob):
                pltpu.make_async_copy(
                    x_hbm.at[pl.ds(ch * _CHUNK, _CHUNK)], xb, sxb).wait()

                @pl.when(i > 0)
                def _drain():
                    pltpu.make_async_copy(
                        ob, out_hbm.at[pl.ds(0, _OUT_CHUNK)], sob).wait()

                _compute(xb, ob)
                pltpu.async_copy(
                    ob, out_hbm.at[pl.ds(ch * _OUT_CHUNK, _OUT_CHUNK)], sob)

                ch2 = ch + 2 * _NW

                @pl.when(ch2 < _N_CHUNKS)
                def _prefetch():
                    pltpu.async_copy(
                        x_hbm.at[pl.ds(ch2 * _CHUNK, _CHUNK)], xb, sxb)
        return c
    lax.fori_loop(0, _ROUNDS // 2, _round_pair, 0)

    for p in range(2):
        ch = wid + (_ROUNDS - 2 + p) * _NW

        @pl.when(ch < _N_CHUNKS)
        def _final(ob=obufs[p], sob=osems[p]):
            pltpu.make_async_copy(
                ob, out_hbm.at[pl.ds(0, _OUT_CHUNK)], sob).wait()


_sc_call = pl.kernel(
    _tec_body,
    out_type=jax.ShapeDtypeStruct((_B * _NUM_BASIS,), jnp.float32),
    mesh=plsc.VectorSubcoreMesh(core_axis_name="c", subcore_axis_name="s"),
    compiler_params=pltpu.CompilerParams(needs_layout_passes=False),
    scratch_types=[
        pltpu.VMEM((_NUM_POINTS * _NUM_BASIS,), jnp.float32),
        pltpu.VMEM((_NUM_POINTS,), jnp.float32),
        pltpu.VMEM((16,), jnp.float32),
        pltpu.VMEM((_CHUNK,), jnp.float32),
        pltpu.VMEM((_CHUNK,), jnp.float32),
        pltpu.VMEM((_OUT_CHUNK,), jnp.float32),
        pltpu.VMEM((_OUT_CHUNK,), jnp.float32),
        pltpu.SemaphoreType.DMA,
        pltpu.SemaphoreType.DMA,
        pltpu.SemaphoreType.DMA,
        pltpu.SemaphoreType.DMA,
    ],
)


def kernel(x, bessel_weights, r_values, bessel_values):
    w2 = jnp.concatenate([bessel_weights, bessel_weights])
    tbl = bessel_values.reshape(-1)
    out = _sc_call(x, w2, r_values, tbl)
    return out.reshape(_B // 128, _NUM_BASIS, 128).transpose(0, 2, 1).reshape(
        _B, _NUM_BASIS)


# speedup vs baseline: 4481.0010x; 5.1028x over previous
import jax
import jax.numpy as jnp
from jax import lax
from jax.experimental import pallas as pl
from jax.experimental.pallas import tpu as pltpu
from jax.experimental.pallas import tpu_sc as plsc

_NUM_POINTS = 5000
_NUM_BASIS = 8
_R_MAX = 5.0
_B = 6400000

_NC = 2
_NS = 16
_NW = _NC * _NS
_CHUNK = 2048
_OUT_CHUNK = _CHUNK * _NUM_BASIS
_N_CHUNKS = _B // _CHUNK
_ROUNDS = -(-_N_CHUNKS // _NW)
assert _ROUNDS % 2 == 0


def _tec_body(x_hbm, w2_hbm, r_hbm, tbl_hbm, out_hbm,
              tbl_v, r_v, w_v, x_v0, x_v1, out_v0, out_v1,
              sx0, sx1, so0, so1):
    wid = lax.axis_index("s") * _NC + lax.axis_index("c")

    pltpu.sync_copy(tbl_hbm, tbl_v)
    pltpu.sync_copy(r_hbm, r_v)
    pltpu.sync_copy(w2_hbm, w_v)

    wpat = w_v[...]
    zero_i = jnp.zeros((16,), jnp.int32)
    one_i = jnp.ones((16,), jnp.int32)
    kmax_i = jnp.full((16,), _NUM_POINTS - 1, jnp.int32)
    nb_i = jnp.full((16,), _NUM_BASIS, jnp.int32)
    rmax_v = jnp.full((16,), _R_MAX, jnp.float32)
    inv_v = jnp.full((16,), (_NUM_POINTS - 1) / _R_MAX, jnp.float32)
    col_iv = [jnp.full((16,), col, jnp.int32) for col in range(_NUM_BASIS)]

    def _scale(i, c):
        sl = pl.ds(i * 16, 16)
        tbl_v[sl] = tbl_v[sl] * wpat
        return c
    lax.fori_loop(0, _NUM_POINTS * _NUM_BASIS // 16, _scale, 0, unroll=4)

    xbufs = (x_v0, x_v1)
    obufs = (out_v0, out_v1)
    xsems = (sx0, sx1)
    osems = (so0, so1)

    def _compute(xb, ob):
        @plsc.parallel_loop(0, _CHUNK // 16, unroll=4)
        def _group(g):
            xv = xb[pl.ds(g * 16, 16)]
            xc = jnp.minimum(xv, rmax_v)
            t = xc * inv_v
            kt = t.astype(jnp.int32)
            ktf = kt.astype(jnp.float32)
            k = kt + jnp.where(ktf < t, one_i, zero_i)
            k = jnp.minimum(jnp.maximum(k, zero_i), kmax_i)
            a = plsc.load_gather(r_v, [k])
            km = jnp.maximum(k - one_i, zero_i)
            b = plsc.load_gather(r_v, [km])
            kup = jnp.minimum(k + one_i, kmax_i)
            down = (b >= xc) & (k > zero_i)
            k = jnp.where(a < xc, kup, jnp.where(down, km, k))
            idx8 = k * nb_i
            vals = [plsc.load_gather(tbl_v, [idx8 + col_iv[col]])
                    for col in range(_NUM_BASIS)]
            off = (g // 8) * (_NUM_BASIS * 128) + (g % 8) * 16
            for col in range(_NUM_BASIS):
                ob[pl.ds(off + col * 128, 16)] = vals[col]

    pltpu.async_copy(x_hbm.at[pl.ds(wid * _CHUNK, _CHUNK)], x_v0, sx0)
    pltpu.async_copy(x_hbm.at[pl.ds((wid + _NW) * _CHUNK, _CHUNK)],
                     x_v1, sx1)

    def _round_pair(i, c):
        for p in range(2):
            r = i * 2 + p
            ch = wid + r * _NW
            xb, ob, sxb, sob = xbufs[p], obufs[p], xsems[p], osems[p]

            @pl.when(ch < _N_CHUNKS)
            def _do(ch=ch, xb=xb, ob=ob, sxb=sxb, sob=sob):
                pltpu.make_async_copy(
                    x_hbm.at[pl.ds(ch * _CHUNK, _CHUNK)], xb, sxb).wait()

                @pl.when(i > 0)
                def _drain():
                    pltpu.make_async_copy(
                        ob, out_hbm.at[pl.ds(0, _OUT_CHUNK)], sob).wait()

                _compute(xb, ob)
                pltpu.async_copy(
                    ob, out_hbm.at[pl.ds(ch * _OUT_CHUNK, _OUT_CHUNK)], sob)

                ch2 = ch + 2 * _NW

                @pl.when(ch2 < _N_CHUNKS)
                def _prefetch():
                    pltpu.async_copy(
                        x_hbm.at[pl.ds(ch2 * _CHUNK, _CHUNK)], xb, sxb)
        return c
    lax.fori_loop(0, _ROUNDS // 2, _round_pair, 0)

    for p in range(2):
        ch = wid + (_ROUNDS - 2 + p) * _NW

        @pl.when(ch < _N_CHUNKS)
        def _final(ob=obufs[p], sob=osems[p]):
            pltpu.make_async_copy(
                ob, out_hbm.at[pl.ds(0, _OUT_CHUNK)], sob).wait()


_sc_call = pl.kernel(
    _tec_body,
    out_type=jax.ShapeDtypeStruct((_B * _NUM_BASIS,), jnp.float32),
    mesh=plsc.VectorSubcoreMesh(core_axis_name="c", subcore_axis_name="s"),
    compiler_params=pltpu.CompilerParams(needs_layout_passes=False),
    scratch_types=[
        pltpu.VMEM((_NUM_POINTS * _NUM_BASIS,), jnp.float32),
        pltpu.VMEM((_NUM_POINTS,), jnp.float32),
        pltpu.VMEM((16,), jnp.float32),
        pltpu.VMEM((_CHUNK,), jnp.float32),
        pltpu.VMEM((_CHUNK,), jnp.float32),
        pltpu.VMEM((_OUT_CHUNK,), jnp.float32),
        pltpu.VMEM((_OUT_CHUNK,), jnp.float32),
        pltpu.SemaphoreType.DMA,
        pltpu.SemaphoreType.DMA,
        pltpu.SemaphoreType.DMA,
        pltpu.SemaphoreType.DMA,
    ],
)


def kernel(x, bessel_weights, r_values, bessel_values):
    w2 = jnp.concatenate([bessel_weights, bessel_weights])
    tbl = bessel_values.reshape(-1)
    out = _sc_call(x, w2, r_values, tbl)
    return out.reshape(_B // 128, _NUM_BASIS, 128).transpose(0, 2, 1).reshape(
        _B, _NUM_BASIS)
